# speedup vs baseline: 28.4871x; 28.4871x over previous
import jax
import jax.numpy as jnp
from jax.experimental import pallas as pl

B = 8
NN = 32
D = 5
C = 32
TOT = C * C * C


def _weights_body(dp0_ref, dp1_ref, dp2_ref, ct0_ref, ct1_ref, ct2_ref, w_ref):
    b = pl.program_id(0)

    def cp(dp_ref):
        d = dp_ref[0]
        lo = d[:, :, 0]
        hi = d[:, :, 1]
        diff = hi - lo
        c_iota = jax.lax.broadcasted_iota(jnp.int32, (NN, C), 1)
        acc = jnp.zeros((NN, C), jnp.float32)
        for dd in range(D):
            bit = ((c_iota >> (D - 1 - dd)) & 1).astype(jnp.float32)
            acc = acc + lo[dd][:, None] + bit * diff[dd][:, None]
        return acc

    cp0 = cp(dp0_ref)
    cp1 = cp(dp1_ref)
    cp2 = cp(dp2_ref)
    s01 = cp0[:, :, None] + cp1[:, None, :]
    w_ref[0] = s01[:, :, :, None] + cp2[:, None, None, :]

    @pl.when(b == 0)
    def _():
        t = jax.lax.broadcasted_iota(jnp.int32, (D, TOT), 1)
        d = jax.lax.broadcasted_iota(jnp.int32, (D, TOT), 0)
        ct0_ref[...] = (t >> (14 - d)) & 1
        ct1_ref[...] = (t >> (9 - d)) & 1
        ct2_ref[...] = (t >> (4 - d)) & 1


def kernel(dp0, dp1, dp2):
    dp_spec = pl.BlockSpec((1, D, NN, 2), lambda b: (b, 0, 0, 0))
    ct_spec = pl.BlockSpec((D, TOT), lambda b: (0, 0))
    out = pl.pallas_call(
        _weights_body,
        grid=(B,),
        in_specs=[dp_spec, dp_spec, dp_spec],
        out_specs=[
            ct_spec,
            ct_spec,
            ct_spec,
            pl.BlockSpec((1, NN, C, C, C), lambda b: (b, 0, 0, 0, 0)),
        ],
        out_shape=[
            jax.ShapeDtypeStruct((D, TOT), jnp.int32),
            jax.ShapeDtypeStruct((D, TOT), jnp.int32),
            jax.ShapeDtypeStruct((D, TOT), jnp.int32),
            jax.ShapeDtypeStruct((B, NN, C, C, C), jnp.float32),
        ],
    )(dp0, dp1, dp2)
    return tuple(out)
